# Initial kernel scaffold; baseline (speedup 1.0000x reference)
import jax
import jax.numpy as jnp
from jax.experimental import pallas as pl


def kernel(x, emb, gamma, beta):
    raise NotImplementedError("write your pallas kernel here")


# speedup vs baseline: 2.0407x; 2.0407x over previous
import jax
import jax.numpy as jnp
from jax.experimental import pallas as pl

EPS = 1e-5


def _ln_body(x_ref, emb_ref, gamma_ref, beta_ref, o_ref):
    h = x_ref[0] + emb_ref[...]
    mean = jnp.mean(h, axis=-1, keepdims=True)
    d = h - mean
    var = jnp.mean(d * d, axis=-1, keepdims=True)
    o_ref[0] = d * jax.lax.rsqrt(var + EPS) * gamma_ref[...] + beta_ref[...]


def kernel(x, emb, gamma, beta):
    B, S, F = x.shape
    Sblk = 512
    grid = (S // Sblk, B)
    return pl.pallas_call(
        _ln_body,
        grid=grid,
        in_specs=[
            pl.BlockSpec((1, Sblk, F), lambda i, j: (j, i, 0)),
            pl.BlockSpec((Sblk, F), lambda i, j: (i, 0)),
            pl.BlockSpec((F,), lambda i, j: (0,)),
            pl.BlockSpec((F,), lambda i, j: (0,)),
        ],
        out_specs=pl.BlockSpec((1, Sblk, F), lambda i, j: (j, i, 0)),
        out_shape=jax.ShapeDtypeStruct((B, S, F), x.dtype),
    )(x, emb[:S], gamma, beta)
